# baseline (speedup 1.0000x reference)
import functools

import jax
import jax.numpy as jnp
from jax.experimental import pallas as pl
from jax.experimental.pallas import tpu as pltpu

IN_DIM = 768
E_DIM = 64
NUM_LEVELS = 3
K = 1024
BETA = 0.25
BATCH = 16384
TILE = 512

_HI = jax.lax.Precision.HIGHEST


def _dot(a, b, precision=jax.lax.Precision.DEFAULT):
    return jax.lax.dot_general(a, b, (((1,), (0,)), ((), ())),
                               precision=precision,
                               preferred_element_type=jnp.float32)


def _rqvae_kernel(x_ref, ew0, eb0, ew1, eb1, ew2, eb2, cb_ref,
                  dw0, db0, dw1, db1, dw2, db2,
                  out_ref, idx_ref, loss_ref):
    i = pl.program_id(0)

    h = jnp.maximum(_dot(x_ref[...], ew0[...]) + eb0[...], 0.0)
    h = jnp.maximum(_dot(h, ew1[...]) + eb1[...], 0.0)
    res = _dot(h, ew2[...]) + eb2[...]

    kiota = jax.lax.broadcasted_iota(jnp.int32, (TILE, K), 1)
    xq_acc = jnp.zeros_like(res)
    loss_sums = []
    idx_cols = []
    for lvl in range(NUM_LEVELS):
        cb = cb_ref[lvl]
        d = ((jnp.sum(res * res, axis=1, keepdims=True)
              - 2.0 * _dot(res, cb.T))
             + jnp.sum(cb * cb, axis=1)[None, :])
        m = jnp.min(d, axis=1, keepdims=True)
        idx = jnp.min(jnp.where(d == m, kiota, K), axis=1)
        onehot = jnp.where(kiota == idx[:, None], 1.0, 0.0)
        xq = _dot(onehot, cb, _HI)
        diff = xq - res
        loss_sums.append(jnp.sum(diff * diff))
        xq_acc = xq_acc + xq
        res = diff * -1.0 if lvl == NUM_LEVELS - 1 else res - xq
        idx_cols.append(idx)

    h = jnp.maximum(_dot(xq_acc, dw0[...]) + db0[...], 0.0)
    h = jnp.maximum(_dot(h, dw1[...]) + db1[...], 0.0)
    out_ref[...] = jax.nn.sigmoid(_dot(h, dw2[...]) + db2[...])
    idx_ref[...] = jnp.stack(idx_cols, axis=1)

    @pl.when(i == 0)
    def _():
        loss_ref[...] = jnp.zeros_like(loss_ref)
    loss_ref[0, :] += jnp.stack(loss_sums)


@functools.partial(jax.jit, static_argnames=())
def _run(x, enc_W0, enc_b0, enc_W1, enc_b1, enc_W2, enc_b2,
         codebooks, dec_W0, dec_b0, dec_W1, dec_b1, dec_W2, dec_b2):
    grid = BATCH // TILE
    full = lambda shape: pl.BlockSpec(shape, lambda i: (0,) * len(shape))
    out, idx, loss = pl.pallas_call(
        _rqvae_kernel,
        grid=(grid,),
        in_specs=[
            pl.BlockSpec((TILE, IN_DIM), lambda i: (i, 0)),
            full(enc_W0.shape), full((1, enc_b0.shape[0])),
            full(enc_W1.shape), full((1, enc_b1.shape[0])),
            full(enc_W2.shape), full((1, enc_b2.shape[0])),
            full(codebooks.shape),
            full(dec_W0.shape), full((1, dec_b0.shape[0])),
            full(dec_W1.shape), full((1, dec_b1.shape[0])),
            full(dec_W2.shape), full((1, dec_b2.shape[0])),
        ],
        out_specs=[
            pl.BlockSpec((TILE, IN_DIM), lambda i: (i, 0)),
            pl.BlockSpec((TILE, NUM_LEVELS), lambda i: (i, 0)),
            pl.BlockSpec((1, NUM_LEVELS), lambda i: (0, 0)),
        ],
        out_shape=[
            jax.ShapeDtypeStruct((BATCH, IN_DIM), jnp.float32),
            jax.ShapeDtypeStruct((BATCH, NUM_LEVELS), jnp.int32),
            jax.ShapeDtypeStruct((1, NUM_LEVELS), jnp.float32),
        ],
    )(x, enc_W0, enc_b0.reshape(1, -1), enc_W1, enc_b1.reshape(1, -1),
      enc_W2, enc_b2.reshape(1, -1), codebooks,
      dec_W0, dec_b0.reshape(1, -1), dec_W1, dec_b1.reshape(1, -1),
      dec_W2, dec_b2.reshape(1, -1))
    per_level_mse = loss[0] / (BATCH * E_DIM)
    rq_loss = jnp.mean((1.0 + BETA) * per_level_mse)
    return out, rq_loss, idx


def kernel(x, epoch_idx, enc_W0, enc_b0, enc_W1, enc_b1, enc_W2, enc_b2,
           codebooks, dec_W0, dec_b0, dec_W1, dec_b1, dec_W2, dec_b2):
    return _run(x, enc_W0, enc_b0, enc_W1, enc_b1, enc_W2, enc_b2,
                codebooks, dec_W0, dec_b0, dec_W1, dec_b1, dec_W2, dec_b2)


# speedup vs baseline: 2.7038x; 2.7038x over previous
import functools

import jax
import jax.numpy as jnp
from jax.experimental import pallas as pl

IN_DIM = 768
E_DIM = 64
NUM_LEVELS = 3
K = 1024
BETA = 0.25
BATCH = 16384
TILE = 512
_CHUNK = 128

_DN = lambda lc, rc: ((lc, rc), ((), ()))


def _dot(a, b, dims=(((1,), (0,)), ((), ()))):
    return jax.lax.dot_general(a, b, dims,
                               precision=jax.lax.Precision.DEFAULT,
                               preferred_element_type=jnp.float32)


def _gather_rows(cbT, idx):
    lo = jnp.bitwise_and(idx, _CHUNK - 1)
    hi = jnp.right_shift(idx, 7)
    lo_b = jax.lax.broadcast_in_dim(lo, (E_DIM, TILE), (1,))
    hi_b = jax.lax.broadcast_in_dim(hi, (E_DIM, TILE), (1,))
    xqT = jnp.zeros((E_DIM, TILE), jnp.float32)
    for h in range(K // _CHUNK):
        g = jnp.take_along_axis(cbT[:, h * _CHUNK:(h + 1) * _CHUNK], lo_b,
                                axis=1)
        xqT = jnp.where(hi_b == h, g, xqT)
    return xqT


def _rqvae_kernel(x_ref, ew0, eb0, ew1, eb1, ew2, eb2, cbT_ref,
                  dw0, db0, dw1, db1, dw2, db2,
                  out_ref, idx_ref, loss_ref):
    i = pl.program_id(0)

    h = jnp.maximum(_dot(x_ref[...], ew0[...]) + eb0[...], 0.0)
    h = jnp.maximum(_dot(h, ew1[...]) + eb1[...], 0.0)
    resT = _dot(ew2[...], h, _DN((0,), (1,))) + eb2[...]

    kiota = jax.lax.broadcasted_iota(jnp.int32, (K, TILE), 0)
    xq_accT = jnp.zeros_like(resT)
    loss_sums = []
    idx_rows = []
    for lvl in range(NUM_LEVELS):
        cbT = cbT_ref[lvl]
        c2 = jnp.sum(cbT * cbT, axis=0)[:, None]
        r2 = jnp.sum(resT * resT, axis=0)[None, :]
        d = (r2 - 2.0 * _dot(cbT, resT, _DN((0,), (0,)))) + c2
        m = jnp.min(d, axis=0, keepdims=True)
        idx = jnp.min(jnp.where(d == m, kiota, K), axis=0)
        xqT = _gather_rows(cbT, idx)
        diffT = xqT - resT
        loss_sums.append(jnp.sum(diffT * diffT))
        xq_accT = xq_accT + xqT
        resT = resT - xqT
        idx_rows.append(idx)

    h = jnp.maximum(_dot(xq_accT, dw0[...], _DN((0,), (0,))) + db0[...], 0.0)
    h = jnp.maximum(_dot(h, dw1[...]) + db1[...], 0.0)
    out_ref[...] = jax.nn.sigmoid(_dot(h, dw2[...]) + db2[...])
    idx_ref[...] = jnp.stack(idx_rows, axis=0)

    @pl.when(i == 0)
    def _():
        loss_ref[...] = jnp.zeros_like(loss_ref)
    loss_ref[...] += jnp.stack(loss_sums)[None, :]


@jax.jit
def _run(x, enc_W0, enc_b0, enc_W1, enc_b1, enc_W2, enc_b2,
         codebooks, dec_W0, dec_b0, dec_W1, dec_b1, dec_W2, dec_b2):
    grid = BATCH // TILE
    full = lambda shape: pl.BlockSpec(shape, lambda i: (0,) * len(shape))
    cbT = codebooks.transpose(0, 2, 1)
    out, idxs, loss = pl.pallas_call(
        _rqvae_kernel,
        grid=(grid,),
        in_specs=[
            pl.BlockSpec((TILE, IN_DIM), lambda i: (i, 0)),
            full(enc_W0.shape), full((1, enc_b0.shape[0])),
            full(enc_W1.shape), full((1, enc_b1.shape[0])),
            full(enc_W2.shape), full((enc_b2.shape[0], 1)),
            full(cbT.shape),
            full(dec_W0.shape), full((1, dec_b0.shape[0])),
            full(dec_W1.shape), full((1, dec_b1.shape[0])),
            full(dec_W2.shape), full((1, dec_b2.shape[0])),
        ],
        out_specs=[
            pl.BlockSpec((TILE, IN_DIM), lambda i: (i, 0)),
            pl.BlockSpec((NUM_LEVELS, TILE), lambda i: (0, i)),
            pl.BlockSpec((1, NUM_LEVELS), lambda i: (0, 0)),
        ],
        out_shape=[
            jax.ShapeDtypeStruct((BATCH, IN_DIM), jnp.float32),
            jax.ShapeDtypeStruct((NUM_LEVELS, BATCH), jnp.int32),
            jax.ShapeDtypeStruct((1, NUM_LEVELS), jnp.float32),
        ],
    )(x, enc_W0, enc_b0.reshape(1, -1), enc_W1, enc_b1.reshape(1, -1),
      enc_W2, enc_b2.reshape(-1, 1), cbT,
      dec_W0, dec_b0.reshape(1, -1), dec_W1, dec_b1.reshape(1, -1),
      dec_W2, dec_b2.reshape(1, -1))
    per_level_mse = loss[0] / (BATCH * E_DIM)
    rq_loss = jnp.mean((1.0 + BETA) * per_level_mse)
    return out, rq_loss, idxs.T


def kernel(x, epoch_idx, enc_W0, enc_b0, enc_W1, enc_b1, enc_W2, enc_b2,
           codebooks, dec_W0, dec_b0, dec_W1, dec_b1, dec_W2, dec_b2):
    return _run(x, enc_W0, enc_b0, enc_W1, enc_b1, enc_W2, enc_b2,
                codebooks, dec_W0, dec_b0, dec_W1, dec_b1, dec_W2, dec_b2)
